# baseline (speedup 1.0000x reference)
import functools

import jax
import jax.numpy as jnp
from jax import lax
from jax.experimental import pallas as pl
from jax.experimental.pallas import tpu as pltpu
from jax.experimental.pallas import tpu_sc as plsc

B = 16384
D = 16
CHUNK = 128


def _sc_gather_body(nc, ns, bpw,
                    uid, iid, tug, tig, tum, tim,
                    oug, oig, oum, oim,
                    xu, xi, bug, big, bum, bim, sem):
  wid = lax.axis_index("s") * nc + lax.axis_index("c")
  base = wid * bpw
  pltpu.sync_copy(uid.at[pl.ds(base, bpw)], xu)
  pltpu.sync_copy(iid.at[pl.ds(base, bpw)], xi)
  copies = []
  for j in range(bpw // CHUNK):
    sl = pl.ds(j * CHUNK, CHUNK)
    copies.append(pltpu.async_copy(tug.at[xu.at[sl]], bug.at[sl], sem))
    copies.append(pltpu.async_copy(tig.at[xi.at[sl]], big.at[sl], sem))
    copies.append(pltpu.async_copy(tum.at[xu.at[sl]], bum.at[sl], sem))
    copies.append(pltpu.async_copy(tim.at[xi.at[sl]], bim.at[sl], sem))
  for c in copies:
    c.wait()
  pltpu.sync_copy(bug, oug.at[pl.ds(base, bpw)])
  pltpu.sync_copy(big, oig.at[pl.ds(base, bpw)])
  pltpu.sync_copy(bum, oum.at[pl.ds(base, bpw)])
  pltpu.sync_copy(bim, oim.at[pl.ds(base, bpw)])


@jax.jit
def _sc_gather(uid, iid, tug, tig, tum, tim):
  info = plsc.get_sparse_core_info()
  nc, ns = info.num_cores, info.num_subcores
  nw = nc * ns
  bpw = B // nw
  mesh = plsc.VectorSubcoreMesh(core_axis_name="c", subcore_axis_name="s")
  row = jax.ShapeDtypeStruct((B, D), jnp.float32)
  body = functools.partial(_sc_gather_body, nc, ns, bpw)
  return pl.kernel(
      body,
      mesh=mesh,
      compiler_params=pltpu.CompilerParams(use_tc_tiling_on_sc=False),
      out_type=(row, row, row, row),
      scratch_types=[
          pltpu.VMEM((bpw,), jnp.int32),
          pltpu.VMEM((bpw,), jnp.int32),
          pltpu.VMEM((bpw, D), jnp.float32),
          pltpu.VMEM((bpw, D), jnp.float32),
          pltpu.VMEM((bpw, D), jnp.float32),
          pltpu.VMEM((bpw, D), jnp.float32),
          pltpu.SemaphoreType.DMA,
      ],
  )(uid, iid, tug, tig, tum, tim)


def _bn_relu(x, g, be):
  mean = jnp.mean(x, axis=0)
  var = jnp.mean((x - mean) ** 2, axis=0)
  x = (x - mean) * lax.rsqrt(var + 1e-5) * g + be
  return jnp.maximum(x, 0.0)


def _tc_body(ug, ig, um, im,
             W0, b0, g0, be0, W1, b1, g1, be1,
             W2, b2, g2, be2, W3, b3, g3, be3,
             Wp, bp, out):
  f32 = jnp.float32
  x = (jnp.dot(um[...], W0[0:D, :], preferred_element_type=f32)
       + jnp.dot(im[...], W0[D:2 * D, :], preferred_element_type=f32)
       + b0[...])
  x = _bn_relu(x, g0[...], be0[...])
  x = jnp.dot(x, W1[...], preferred_element_type=f32) + b1[...]
  x = _bn_relu(x, g1[...], be1[...])
  x = jnp.dot(x, W2[...], preferred_element_type=f32) + b2[...]
  x = _bn_relu(x, g2[...], be2[...])
  x = jnp.dot(x, W3[...], preferred_element_type=f32) + b3[...]
  x = _bn_relu(x, g3[...], be3[...])
  gmf = ug[...] * ig[...]
  logit = (jnp.dot(gmf, Wp[0:D, :], preferred_element_type=f32)
           + jnp.dot(x, Wp[D:D + 8, :], preferred_element_type=f32)
           + bp[...])
  out[...] = jax.nn.sigmoid(logit)


@jax.jit
def _tc_mlp(ug, ig, um, im, *weights):
  return pl.pallas_call(
      _tc_body,
      out_shape=jax.ShapeDtypeStruct((B, 1), jnp.float32),
  )(ug, ig, um, im, *weights)


def kernel(user_indices, item_indices, user_gmf, item_gmf, user_mlp, item_mlp,
           W0, b0, g0, be0, W1, b1, g1, be1, W2, b2, g2, be2, W3, b3, g3, be3,
           Wp, bp):
  uid = user_indices.astype(jnp.int32)
  iid = item_indices.astype(jnp.int32)
  ug, ig, um, im = _sc_gather(uid, iid, user_gmf, item_gmf, user_mlp, item_mlp)
  pred = _tc_mlp(ug, ig, um, im,
                 W0, b0, g0, be0, W1, b1, g1, be1,
                 W2, b2, g2, be2, W3, b3, g3, be3, Wp, bp)
  return jnp.squeeze(pred, axis=-1)
